# baseline (device time: 54458 ns/iter reference)
import jax
import jax.numpy as jnp
from jax import lax
from jax.experimental import pallas as pl
from jax.experimental.pallas import tpu as pltpu


def kernel(Q, K, V):
    b, s, h, d = Q.shape
    bh = b * h
    scale = d ** -0.5

    def to_bhsd(x):
        return x.transpose(0, 2, 1, 3).reshape(bh, s, d)

    Qt, Kt, Vt = to_bhsd(Q), to_bhsd(K), to_bhsd(V)

    def body(q_ref, k_ref, v_ref, o_ref, ko_ref, vo_ref, send_sems, recv_sems):
        my_x = lax.axis_index("x")
        my_y = lax.axis_index("y")
        my_z = lax.axis_index("z")
        peer = (1 - my_x, my_y, my_z)

        barrier = pltpu.get_barrier_semaphore()
        pl.semaphore_signal(
            barrier, inc=1, device_id=peer,
            device_id_type=pl.DeviceIdType.MESH,
        )
        pl.semaphore_wait(barrier, 1)

        k_rdma = pltpu.make_async_remote_copy(
            src_ref=k_ref, dst_ref=ko_ref,
            send_sem=send_sems.at[0], recv_sem=recv_sems.at[0],
            device_id=peer, device_id_type=pl.DeviceIdType.MESH,
        )
        v_rdma = pltpu.make_async_remote_copy(
            src_ref=v_ref, dst_ref=vo_ref,
            send_sem=send_sems.at[1], recv_sem=recv_sems.at[1],
            device_id=peer, device_id_type=pl.DeviceIdType.MESH,
        )
        k_rdma.start()
        v_rdma.start()
        k_rdma.wait()
        v_rdma.wait()
        o_ref[...] = ko_ref[...]

    out = pl.pallas_call(
        body,
        out_shape=jax.ShapeDtypeStruct((bh, s, d), jnp.float32),
        in_specs=[pl.BlockSpec(memory_space=pltpu.VMEM)] * 3,
        out_specs=pl.BlockSpec(memory_space=pltpu.VMEM),
        scratch_shapes=[
            pltpu.VMEM((bh, s, d), jnp.float32),
            pltpu.VMEM((bh, s, d), jnp.float32),
            pltpu.SemaphoreType.DMA((2,)),
            pltpu.SemaphoreType.DMA((2,)),
        ],
        compiler_params=pltpu.CompilerParams(collective_id=0),
    )(Qt, Kt, Vt)

    return out.reshape(b, h, s, d).transpose(0, 2, 1, 3)


# device time: 33146 ns/iter; 1.6430x vs baseline; 1.6430x over previous
import jax
import jax.numpy as jnp
from jax import lax
from jax.experimental import pallas as pl
from jax.experimental.pallas import tpu as pltpu


def kernel(Q, K, V):
    b, s, h, d = Q.shape
    bh = b * h
    scale = d ** -0.5

    Qt = Q.transpose(0, 2, 1, 3).reshape(bh, s, d)
    Kt = K.transpose(0, 2, 3, 1).reshape(bh, d, s)
    Vt = V.transpose(0, 2, 3, 1).reshape(bh, d, s)

    def body(q_ref, k_ref, v_ref, o_ref, ko_ref, vo_ref, send_sems, recv_sems):
        my_x = lax.axis_index("x")
        my_y = lax.axis_index("y")
        my_z = lax.axis_index("z")
        peer = (1 - my_x, my_y, my_z)

        barrier = pltpu.get_barrier_semaphore()
        pl.semaphore_signal(
            barrier, inc=1, device_id=peer,
            device_id_type=pl.DeviceIdType.MESH,
        )
        pl.semaphore_wait(barrier, 1)

        k_rdma = pltpu.make_async_remote_copy(
            src_ref=k_ref, dst_ref=ko_ref,
            send_sem=send_sems.at[0], recv_sem=recv_sems.at[0],
            device_id=peer, device_id_type=pl.DeviceIdType.MESH,
        )
        v_rdma = pltpu.make_async_remote_copy(
            src_ref=v_ref, dst_ref=vo_ref,
            send_sem=send_sems.at[1], recv_sem=recv_sems.at[1],
            device_id=peer, device_id_type=pl.DeviceIdType.MESH,
        )
        k_rdma.start()
        v_rdma.start()
        k_rdma.wait()
        v_rdma.wait()

        for i in range(bh):
            q = q_ref[i]
            kl = k_ref[i]
            kr = ko_ref[i]
            s_l = lax.dot_general(
                q, kl, (((1,), (0,)), ((), ()))) * scale
            s_r = lax.dot_general(
                q, kr, (((1,), (0,)), ((), ()))) * scale
            m = jnp.maximum(
                jnp.max(s_l, axis=1, keepdims=True),
                jnp.max(s_r, axis=1, keepdims=True),
            )
            p_l = jnp.exp(s_l - m)
            p_r = jnp.exp(s_r - m)
            denom = (jnp.sum(p_l, axis=1, keepdims=True)
                     + jnp.sum(p_r, axis=1, keepdims=True))
            acc = (lax.dot_general(p_l, v_ref[i], (((1,), (1,)), ((), ())))
                   + lax.dot_general(p_r, vo_ref[i], (((1,), (1,)), ((), ()))))
            o_ref[i] = acc / denom

    out = pl.pallas_call(
        body,
        out_shape=jax.ShapeDtypeStruct((bh, s, d), jnp.float32),
        in_specs=[pl.BlockSpec(memory_space=pltpu.VMEM)] * 3,
        out_specs=pl.BlockSpec(memory_space=pltpu.VMEM),
        scratch_shapes=[
            pltpu.VMEM((bh, d, s), jnp.float32),
            pltpu.VMEM((bh, d, s), jnp.float32),
            pltpu.SemaphoreType.DMA((2,)),
            pltpu.SemaphoreType.DMA((2,)),
        ],
        compiler_params=pltpu.CompilerParams(collective_id=0),
    )(Qt, Kt, Vt)

    return out.reshape(b, h, s, d).transpose(0, 2, 1, 3)


# device time: 27301 ns/iter; 1.9947x vs baseline; 1.2141x over previous
import jax
import jax.numpy as jnp
from jax import lax
from jax.experimental import pallas as pl
from jax.experimental.pallas import tpu as pltpu


def kernel(Q, K, V):
    b, s, h, d = Q.shape
    bh = b * h
    hq = bh // 4
    scale = d ** -0.5

    Qt = Q.transpose(0, 2, 1, 3).reshape(bh, s, d)
    Kt = K.transpose(0, 2, 3, 1).reshape(bh, d, s)
    Vt = V.transpose(0, 2, 3, 1).reshape(bh, d, s)

    def body(q_ref, k_ref, v_ref, o_ref, rk, rv, send_sems, recv_sems):
        my_x = lax.axis_index("x")
        my_y = lax.axis_index("y")
        my_z = lax.axis_index("z")
        peer_x = (1 - my_x, my_y, my_z)
        nb_y = (my_x, 1 - my_y, my_z)
        nb_z = (my_x, my_y, 1 - my_z)

        j_me = 2 * my_y + my_z
        j_y = 2 * (1 - my_y) + my_z
        j_z = 2 * my_y + (1 - my_z)
        j_diag = 2 * (1 - my_y) + (1 - my_z)

        def ksl(ref, j):
            return ref.at[pl.ds(hq * j, hq)]

        barrier = pltpu.get_barrier_semaphore()
        for nbr in (peer_x, nb_y, nb_z):
            pl.semaphore_signal(barrier, inc=1, device_id=nbr,
                                device_id_type=pl.DeviceIdType.MESH)
        pl.semaphore_wait(barrier, 3)

        def copy(src, dst, sem_i, dev):
            return pltpu.make_async_remote_copy(
                src_ref=src, dst_ref=dst,
                send_sem=send_sems.at[sem_i], recv_sem=recv_sems.at[sem_i],
                device_id=dev, device_id_type=pl.DeviceIdType.MESH,
            )

        o1k = copy(ksl(k_ref, j_me), ksl(rk, j_me), 0, peer_x)
        o1v = copy(ksl(v_ref, j_me), ksl(rv, j_me), 1, peer_x)
        o2ky = copy(ksl(rk, j_me), ksl(rk, j_me), 2, nb_y)
        o2vy = copy(ksl(rv, j_me), ksl(rv, j_me), 3, nb_y)
        o2kz = copy(ksl(rk, j_me), ksl(rk, j_me), 4, nb_z)
        o2vz = copy(ksl(rv, j_me), ksl(rv, j_me), 5, nb_z)
        o3y = copy(ksl(rk, j_z), ksl(rk, j_z), 6, nb_y)
        o3z = copy(ksl(rv, j_y), ksl(rv, j_y), 7, nb_z)

        i2k = copy(ksl(k_ref, j_me), ksl(rk, j_y), 2, nb_y)
        i2v = copy(ksl(v_ref, j_me), ksl(rv, j_y), 3, nb_y)
        i3k = copy(ksl(k_ref, j_me), ksl(rk, j_z), 4, nb_z)
        i3v = copy(ksl(v_ref, j_me), ksl(rv, j_z), 5, nb_z)
        i4k = copy(ksl(k_ref, j_me), ksl(rk, j_diag), 6, nb_y)
        i4v = copy(ksl(v_ref, j_me), ksl(rv, j_diag), 7, nb_z)

        def compute_quarter(j):
            for t in range(hq):
                i = hq * j + t
                q = q_ref[i]
                s_l = lax.dot_general(
                    q, k_ref[i], (((1,), (0,)), ((), ()))) * scale
                s_r = lax.dot_general(
                    q, rk[i], (((1,), (0,)), ((), ()))) * scale
                m = jnp.maximum(
                    jnp.max(s_l, axis=1, keepdims=True),
                    jnp.max(s_r, axis=1, keepdims=True),
                )
                p_l = jnp.exp(s_l - m)
                p_r = jnp.exp(s_r - m)
                denom = (jnp.sum(p_l, axis=1, keepdims=True)
                         + jnp.sum(p_r, axis=1, keepdims=True))
                acc = (lax.dot_general(p_l, v_ref[i], (((1,), (1,)), ((), ())))
                       + lax.dot_general(p_r, rv[i], (((1,), (1,)), ((), ()))))
                o_ref[i] = acc / denom

        o1k.start()
        o1v.start()
        o1k.wait_recv()
        o1v.wait_recv()
        o2ky.start()
        o2vy.start()
        o2kz.start()
        o2vz.start()
        compute_quarter(j_me)
        i3k.wait_recv()
        o3y.start()
        i2v.wait_recv()
        o3z.start()
        i2k.wait_recv()
        compute_quarter(j_y)
        i3v.wait_recv()
        compute_quarter(j_z)
        i4k.wait_recv()
        i4v.wait_recv()
        compute_quarter(j_diag)

        for dsc in (o1k, o1v, o2ky, o2vy, o2kz, o2vz, o3y, o3z):
            dsc.wait_send()

    out = pl.pallas_call(
        body,
        out_shape=jax.ShapeDtypeStruct((bh, s, d), jnp.float32),
        in_specs=[pl.BlockSpec(memory_space=pltpu.VMEM)] * 3,
        out_specs=pl.BlockSpec(memory_space=pltpu.VMEM),
        scratch_shapes=[
            pltpu.VMEM((bh, d, s), jnp.float32),
            pltpu.VMEM((bh, d, s), jnp.float32),
            pltpu.SemaphoreType.DMA((8,)),
            pltpu.SemaphoreType.DMA((8,)),
        ],
        compiler_params=pltpu.CompilerParams(collective_id=0),
    )(Qt, Kt, Vt)

    return out.reshape(b, h, s, d).transpose(0, 2, 1, 3)


# device time: 22374 ns/iter; 2.4340x vs baseline; 1.2202x over previous
import jax
import jax.numpy as jnp
from jax import lax
from jax.experimental import pallas as pl
from jax.experimental.pallas import tpu as pltpu


def kernel(Q, K, V):
    b, s, h, d = Q.shape
    bh = b * h
    hq = bh // 4
    hc = hq // 2
    scale = d ** -0.5

    Qt = Q.transpose(0, 2, 1, 3).reshape(bh, s, d)
    Kt = K.transpose(0, 2, 3, 1).reshape(bh, d, s)
    Vt = V.transpose(0, 2, 3, 1).reshape(bh, d, s)

    def body(q_hbm, k_hbm, v_hbm, o_ref, qv, kv, vv, rk, rv,
             lsem, send_sems, recv_sems):
        my_x = lax.axis_index("x")
        my_y = lax.axis_index("y")
        my_z = lax.axis_index("z")
        peer_x = (1 - my_x, my_y, my_z)
        nb_y = (my_x, 1 - my_y, my_z)
        nb_z = (my_x, my_y, 1 - my_z)

        j_me = 2 * my_y + my_z
        j_y = 2 * (1 - my_y) + my_z
        j_z = 2 * my_y + (1 - my_z)
        j_diag = 2 * (1 - my_y) + (1 - my_z)

        def csl(ref, j, c):
            return ref.at[pl.ds(hq * j + hc * c, hc)]

        cq = pltpu.make_async_copy(q_hbm, qv, lsem.at[0])
        ck = pltpu.make_async_copy(k_hbm, kv, lsem.at[1])
        cv = pltpu.make_async_copy(v_hbm, vv, lsem.at[2])
        cq.start()
        ck.start()
        cv.start()

        barrier = pltpu.get_barrier_semaphore()
        for nbr in (peer_x, nb_y, nb_z):
            pl.semaphore_signal(barrier, inc=1, device_id=nbr,
                                device_id_type=pl.DeviceIdType.MESH)
        pl.semaphore_wait(barrier, 3)

        def copy(src, dst, sem_i, dev):
            return pltpu.make_async_remote_copy(
                src_ref=src, dst_ref=dst,
                send_sem=send_sems.at[sem_i], recv_sem=recv_sems.at[sem_i],
                device_id=dev, device_id_type=pl.DeviceIdType.MESH,
            )

        C = (0, 1)
        o1k = [copy(csl(k_hbm, j_me, c), csl(rk, j_me, c), 0 + c, peer_x)
               for c in C]
        o1v = [copy(csl(v_hbm, j_me, c), csl(rv, j_me, c), 2 + c, peer_x)
               for c in C]
        o2ky = [copy(csl(rk, j_me, c), csl(rk, j_me, c), 4 + c, nb_y)
                for c in C]
        o2vy = [copy(csl(rv, j_me, c), csl(rv, j_me, c), 6 + c, nb_y)
                for c in C]
        o2kz = [copy(csl(rk, j_me, c), csl(rk, j_me, c), 8 + c, nb_z)
                for c in C]
        o2vz = [copy(csl(rv, j_me, c), csl(rv, j_me, c), 10 + c, nb_z)
                for c in C]
        o3y = [copy(csl(rk, j_z, c), csl(rk, j_z, c), 12 + c, nb_y)
               for c in C]
        o3z = [copy(csl(rv, j_y, c), csl(rv, j_y, c), 14 + c, nb_z)
               for c in C]

        dummy = kv.at[pl.ds(0, hc)]
        i2k = [copy(dummy, csl(rk, j_y, c), 4 + c, nb_y) for c in C]
        i2v = [copy(dummy, csl(rv, j_y, c), 6 + c, nb_y) for c in C]
        i3k = [copy(dummy, csl(rk, j_z, c), 8 + c, nb_z) for c in C]
        i3v = [copy(dummy, csl(rv, j_z, c), 10 + c, nb_z) for c in C]
        i4k = [copy(dummy, csl(rk, j_diag, c), 12 + c, nb_y) for c in C]
        i4v = [copy(dummy, csl(rv, j_diag, c), 14 + c, nb_z) for c in C]

        def compute_quarter(j):
            for t in range(hq):
                i = hq * j + t
                q = qv[i]
                s_l = lax.dot_general(
                    q, kv[i], (((1,), (0,)), ((), ()))) * scale
                s_r = lax.dot_general(
                    q, rk[i], (((1,), (0,)), ((), ()))) * scale
                m = jnp.maximum(
                    jnp.max(s_l, axis=1, keepdims=True),
                    jnp.max(s_r, axis=1, keepdims=True),
                )
                p_l = jnp.exp(s_l - m)
                p_r = jnp.exp(s_r - m)
                denom = (jnp.sum(p_l, axis=1, keepdims=True)
                         + jnp.sum(p_r, axis=1, keepdims=True))
                acc = (lax.dot_general(p_l, vv[i], (((1,), (1,)), ((), ())))
                       + lax.dot_general(p_r, rv[i], (((1,), (1,)), ((), ()))))
                o_ref[i] = acc / denom

        for c in C:
            o1k[c].start()
            o1v[c].start()
        for c in C:
            o1k[c].wait_recv()
            o2ky[c].start()
            o2kz[c].start()
            o1v[c].wait_recv()
            o2vy[c].start()
            o2vz[c].start()
        cq.wait()
        ck.wait()
        cv.wait()
        compute_quarter(j_me)
        i3k[0].wait_recv()
        o3y[0].start()
        i2v[0].wait_recv()
        o3z[0].start()
        i3k[1].wait_recv()
        o3y[1].start()
        i2v[1].wait_recv()
        o3z[1].start()
        i2k[0].wait_recv()
        i2k[1].wait_recv()
        compute_quarter(j_y)
        i3v[0].wait_recv()
        i3v[1].wait_recv()
        compute_quarter(j_z)
        for c in C:
            i4k[c].wait_recv()
            i4v[c].wait_recv()
        compute_quarter(j_diag)

        for dsc in (o1k + o1v + o2ky + o2vy + o2kz + o2vz + o3y + o3z):
            dsc.wait_send()

    out = pl.pallas_call(
        body,
        out_shape=jax.ShapeDtypeStruct((bh, s, d), jnp.float32),
        in_specs=[pl.BlockSpec(memory_space=pltpu.MemorySpace.HBM)] * 3,
        out_specs=pl.BlockSpec(memory_space=pltpu.VMEM),
        scratch_shapes=[
            pltpu.VMEM((bh, s, d), jnp.float32),
            pltpu.VMEM((bh, d, s), jnp.float32),
            pltpu.VMEM((bh, d, s), jnp.float32),
            pltpu.VMEM((bh, d, s), jnp.float32),
            pltpu.VMEM((bh, d, s), jnp.float32),
            pltpu.SemaphoreType.DMA((3,)),
            pltpu.SemaphoreType.DMA((16,)),
            pltpu.SemaphoreType.DMA((16,)),
        ],
        compiler_params=pltpu.CompilerParams(collective_id=0),
    )(Qt, Kt, Vt)

    return out.reshape(b, h, s, d).transpose(0, 2, 1, 3)
